# Initial kernel scaffold; baseline (speedup 1.0000x reference)
import jax
import jax.numpy as jnp
from jax.experimental import pallas as pl


def kernel(x, emb, W1, b1, W2, b2, W3, b3):
    raise NotImplementedError("write your pallas kernel here")


# speedup vs baseline: 7.0541x; 7.0541x over previous
import functools

import jax
import jax.numpy as jnp
from jax import lax
from jax.experimental import pallas as pl
from jax.experimental.pallas import tpu as pltpu
from jax.experimental.pallas import tpu_sc as plsc

BATCH = 16384
EMB_DIM = 256
OTHER_DIM = 32
HIDDEN = 64
OUT_DIM = 64

NC = 2
NS = 16
NW = NC * NS
NIDX = 2 * BATCH
BPW = NIDX // NW
CHUNK = 128
NCHUNK = BPW // CHUNK


def _make_sc_gather(vocab: int):
    mesh = plsc.VectorSubcoreMesh(core_axis_name="c", subcore_axis_name="s")

    @functools.partial(
        pl.kernel,
        mesh=mesh,
        out_type=jax.ShapeDtypeStruct((NIDX, EMB_DIM), jnp.float32),
        scratch_types=[
            pltpu.VMEM((CHUNK,), jnp.int32),
            pltpu.VMEM((CHUNK, EMB_DIM), jnp.float32),
            pltpu.SemaphoreType.DMA,
        ],
    )
    def gather_k(table_hbm, idx_hbm, out_hbm, idx_v, rows_v, sem):
        wid = lax.axis_index("s") * NC + lax.axis_index("c")
        base = wid * BPW

        def body(i, carry):
            off = base + i * CHUNK
            pltpu.sync_copy(idx_hbm.at[pl.ds(off, CHUNK)], idx_v)
            pltpu.async_copy(table_hbm.at[idx_v], rows_v, sem).wait()
            pltpu.sync_copy(rows_v, out_hbm.at[pl.ds(off, CHUNK)])
            return carry

        lax.fori_loop(0, NCHUNK, body, 0)

    return gather_k


def _mlp_body(g1, g2, o, w1a, w1b, w1c, b1, w2, b2, w3, b3, out):
    h = (
        jnp.dot(g1[...], w1a[...], preferred_element_type=jnp.float32)
        + jnp.dot(g2[...], w1b[...], preferred_element_type=jnp.float32)
        + jnp.dot(o[...], w1c[...], preferred_element_type=jnp.float32)
        + b1[...]
    )
    h = jnp.maximum(h, 0.0)
    h = jnp.maximum(jnp.dot(h, w2[...], preferred_element_type=jnp.float32) + b2[...], 0.0)
    out[...] = jnp.dot(h, w3[...], preferred_element_type=jnp.float32) + b3[...]


BB = 1024
NB = BATCH // BB


def _mlp_call(gathered, other, W1a, W1b, W1c, b1, W2, b2, W3, b3):
    full = lambda shape: pl.BlockSpec(shape, lambda i: (0, 0))
    return pl.pallas_call(
        _mlp_body,
        grid=(NB,),
        in_specs=[
            pl.BlockSpec((BB, EMB_DIM), lambda i: (i, 0)),
            pl.BlockSpec((BB, EMB_DIM), lambda i: (i + NB, 0)),
            pl.BlockSpec((BB, OTHER_DIM), lambda i: (i, 0)),
            full((EMB_DIM, HIDDEN)),
            full((EMB_DIM, HIDDEN)),
            full((OTHER_DIM, HIDDEN)),
            full((1, HIDDEN)),
            full((HIDDEN, HIDDEN)),
            full((1, HIDDEN)),
            full((HIDDEN, OUT_DIM)),
            full((1, OUT_DIM)),
        ],
        out_specs=pl.BlockSpec((BB, OUT_DIM), lambda i: (i, 0)),
        out_shape=jax.ShapeDtypeStruct((BATCH, OUT_DIM), jnp.float32),
    )(gathered, gathered, other, W1a, W1b, W1c, b1, W2, b2, W3, b3)


def kernel(x, emb, W1, b1, W2, b2, W3, b3):
    idx = x[:, :2].astype(jnp.int32)
    idx_all = jnp.concatenate([idx[:, 0], idx[:, 1]], axis=0)
    other = x[:, 2:]

    gathered = _make_sc_gather(emb.shape[0])(emb, idx_all)

    W1a = W1[:EMB_DIM]
    W1b = W1[EMB_DIM : 2 * EMB_DIM]
    W1c = W1[2 * EMB_DIM :]
    return _mlp_call(
        gathered,
        other,
        W1a,
        W1b,
        W1c,
        b1.reshape(1, HIDDEN),
        W2,
        b2.reshape(1, HIDDEN),
        W3,
        b3.reshape(1, OUT_DIM),
    )


# speedup vs baseline: 7.6148x; 1.0795x over previous
import functools

import jax
import jax.numpy as jnp
from jax import lax
from jax.experimental import pallas as pl
from jax.experimental.pallas import tpu as pltpu
from jax.experimental.pallas import tpu_sc as plsc

BATCH = 16384
EMB_DIM = 256
OTHER_DIM = 32
HIDDEN = 64
OUT_DIM = 64

NC = 2
NS = 16
NW = NC * NS
NIDX = 2 * BATCH
BPW = NIDX // NW
CHUNK = 128
NCHUNK = BPW // CHUNK


def _make_sc_gather(vocab: int):
    mesh = plsc.VectorSubcoreMesh(core_axis_name="c", subcore_axis_name="s")

    @functools.partial(
        pl.kernel,
        mesh=mesh,
        out_type=jax.ShapeDtypeStruct((NIDX, EMB_DIM), jnp.float32),
        scratch_types=[
            pltpu.VMEM((BPW,), jnp.int32),
            pltpu.VMEM((CHUNK, EMB_DIM), jnp.float32),
            pltpu.VMEM((CHUNK, EMB_DIM), jnp.float32),
            pltpu.SemaphoreType.DMA,
            pltpu.SemaphoreType.DMA,
        ],
    )
    def gather_k(table_hbm, idx_hbm, out_hbm, idx_v, rows0, rows1, sem_g, sem_o):
        wid = lax.axis_index("s") * NC + lax.axis_index("c")
        base = wid * BPW
        rows = (rows0, rows1)

        pltpu.sync_copy(idx_hbm.at[pl.ds(base, BPW)], idx_v)
        gathers = [None, None]
        outs = [None, None]
        for j in range(NCHUNK):
            b = j % 2
            if j >= 2:
                outs[b].wait()
            gathers[b] = pltpu.async_copy(
                table_hbm.at[idx_v.at[pl.ds(j * CHUNK, CHUNK)]], rows[b], sem_g
            )
            if j >= 1:
                bp = (j - 1) % 2
                gathers[bp].wait()
                outs[bp] = pltpu.async_copy(
                    rows[bp], out_hbm.at[pl.ds(base + (j - 1) * CHUNK, CHUNK)], sem_o
                )
        bl = (NCHUNK - 1) % 2
        gathers[bl].wait()
        outs[bl] = pltpu.async_copy(
            rows[bl], out_hbm.at[pl.ds(base + (NCHUNK - 1) * CHUNK, CHUNK)], sem_o
        )
        outs[1 - bl].wait()
        outs[bl].wait()

    return gather_k


def _mlp_body(g1, g2, o, w1a, w1b, w1c, b1, w2, b2, w3, b3, out):
    h = (
        jnp.dot(g1[...], w1a[...], preferred_element_type=jnp.float32)
        + jnp.dot(g2[...], w1b[...], preferred_element_type=jnp.float32)
        + jnp.dot(o[...], w1c[...], preferred_element_type=jnp.float32)
        + b1[...]
    )
    h = jnp.maximum(h, 0.0)
    h = jnp.maximum(jnp.dot(h, w2[...], preferred_element_type=jnp.float32) + b2[...], 0.0)
    out[...] = jnp.dot(h, w3[...], preferred_element_type=jnp.float32) + b3[...]


BB = 1024
NB = BATCH // BB


def _mlp_call(gathered, other, W1a, W1b, W1c, b1, W2, b2, W3, b3):
    full = lambda shape: pl.BlockSpec(shape, lambda i: (0, 0))
    return pl.pallas_call(
        _mlp_body,
        grid=(NB,),
        in_specs=[
            pl.BlockSpec((BB, EMB_DIM), lambda i: (i, 0)),
            pl.BlockSpec((BB, EMB_DIM), lambda i: (i + NB, 0)),
            pl.BlockSpec((BB, OTHER_DIM), lambda i: (i, 0)),
            full((EMB_DIM, HIDDEN)),
            full((EMB_DIM, HIDDEN)),
            full((OTHER_DIM, HIDDEN)),
            full((1, HIDDEN)),
            full((HIDDEN, HIDDEN)),
            full((1, HIDDEN)),
            full((HIDDEN, OUT_DIM)),
            full((1, OUT_DIM)),
        ],
        out_specs=pl.BlockSpec((BB, OUT_DIM), lambda i: (i, 0)),
        out_shape=jax.ShapeDtypeStruct((BATCH, OUT_DIM), jnp.float32),
    )(gathered, gathered, other, W1a, W1b, W1c, b1, W2, b2, W3, b3)


def kernel(x, emb, W1, b1, W2, b2, W3, b3):
    idx = x[:, :2].astype(jnp.int32)
    idx_all = jnp.concatenate([idx[:, 0], idx[:, 1]], axis=0)
    other = x[:, 2:]

    gathered = _make_sc_gather(emb.shape[0])(emb, idx_all)

    W1a = W1[:EMB_DIM]
    W1b = W1[EMB_DIM : 2 * EMB_DIM]
    W1c = W1[2 * EMB_DIM :]
    return _mlp_call(
        gathered,
        other,
        W1a,
        W1b,
        W1c,
        b1.reshape(1, HIDDEN),
        W2,
        b2.reshape(1, HIDDEN),
        W3,
        b3.reshape(1, OUT_DIM),
    )


# speedup vs baseline: 8.8932x; 1.1679x over previous
import functools

import jax
import jax.numpy as jnp
from jax import lax
from jax.experimental import pallas as pl
from jax.experimental.pallas import tpu as pltpu
from jax.experimental.pallas import tpu_sc as plsc

BATCH = 16384
EMB_DIM = 256
OTHER_DIM = 32
HIDDEN = 64
OUT_DIM = 64

NC = 2
NS = 16
NW = NC * NS
NIDX = 2 * BATCH
BPW = NIDX // NW
CHUNK = 128
NCHUNK = BPW // CHUNK


def _make_sc_gather(vocab: int):
    mesh = plsc.VectorSubcoreMesh(core_axis_name="c", subcore_axis_name="s")

    @functools.partial(
        pl.kernel,
        mesh=mesh,
        out_type=jax.ShapeDtypeStruct((NIDX, EMB_DIM), jnp.float32),
        scratch_types=[
            pltpu.VMEM((BPW,), jnp.int32),
            pltpu.VMEM((CHUNK, EMB_DIM), jnp.float32),
            pltpu.VMEM((CHUNK, EMB_DIM), jnp.float32),
            pltpu.SemaphoreType.DMA,
            pltpu.SemaphoreType.DMA,
        ],
    )
    def gather_k(table_hbm, idx_hbm, out_hbm, idx_v, rows0, rows1, sem_g, sem_o):
        wid = lax.axis_index("s") * NC + lax.axis_index("c")
        base = wid * BPW
        rows = (rows0, rows1)

        pltpu.sync_copy(idx_hbm.at[pl.ds(base, BPW)], idx_v)
        gathers = [None, None]
        outs = [None, None]
        for j in range(NCHUNK):
            b = j % 2
            if j >= 2:
                outs[b].wait()
            gathers[b] = pltpu.async_copy(
                table_hbm.at[idx_v.at[pl.ds(j * CHUNK, CHUNK)]], rows[b], sem_g
            )
            if j >= 1:
                bp = (j - 1) % 2
                gathers[bp].wait()
                outs[bp] = pltpu.async_copy(
                    rows[bp], out_hbm.at[pl.ds(base + (j - 1) * CHUNK, CHUNK)], sem_o
                )
        bl = (NCHUNK - 1) % 2
        gathers[bl].wait()
        outs[bl] = pltpu.async_copy(
            rows[bl], out_hbm.at[pl.ds(base + (NCHUNK - 1) * CHUNK, CHUNK)], sem_o
        )
        outs[1 - bl].wait()
        outs[bl].wait()

    return gather_k


def _mlp_body(g1, g2, ot, w1at, w1bt, w1ct, b1, w2t, b2, w3t, b3, out):
    dn_rt = (((1,), (1,)), ((), ()))
    hg = (
        lax.dot_general(g1[...], w1at[...], dn_rt, preferred_element_type=jnp.float32)
        + lax.dot_general(g2[...], w1bt[...], dn_rt, preferred_element_type=jnp.float32)
        + b1[...]
    )
    h1t = hg.T + jnp.dot(w1ct[...], ot[...], preferred_element_type=jnp.float32)
    h1t = jnp.maximum(h1t, 0.0)
    h2t = jnp.maximum(
        jnp.dot(w2t[...], h1t, preferred_element_type=jnp.float32) + b2[...].T, 0.0
    )
    out[...] = jnp.dot(w3t[...], h2t, preferred_element_type=jnp.float32) + b3[...].T


BB = 1024
NB = BATCH // BB


def _mlp_call(gathered, other_t, W1at, W1bt, W1ct, b1, W2t, b2, W3t, b3):
    full = lambda shape: pl.BlockSpec(shape, lambda i: (0, 0))
    return pl.pallas_call(
        _mlp_body,
        grid=(NB,),
        in_specs=[
            pl.BlockSpec((BB, EMB_DIM), lambda i: (i, 0)),
            pl.BlockSpec((BB, EMB_DIM), lambda i: (i + NB, 0)),
            pl.BlockSpec((OTHER_DIM, BB), lambda i: (0, i)),
            full((HIDDEN, EMB_DIM)),
            full((HIDDEN, EMB_DIM)),
            full((HIDDEN, OTHER_DIM)),
            full((1, HIDDEN)),
            full((HIDDEN, HIDDEN)),
            full((1, HIDDEN)),
            full((OUT_DIM, HIDDEN)),
            full((1, OUT_DIM)),
        ],
        out_specs=pl.BlockSpec((OUT_DIM, BB), lambda i: (0, i)),
        out_shape=jax.ShapeDtypeStruct((OUT_DIM, BATCH), jnp.float32),
    )(gathered, gathered, other_t, W1at, W1bt, W1ct, b1, W2t, b2, W3t, b3)


def kernel(x, emb, W1, b1, W2, b2, W3, b3):
    x_t = x.T
    idx_all = jnp.concatenate([x_t[0], x_t[1]], axis=0).astype(jnp.int32)
    other_t = x_t[2:]

    gathered = _make_sc_gather(emb.shape[0])(emb, idx_all)

    out_t = _mlp_call(
        gathered,
        other_t,
        W1[:EMB_DIM].T,
        W1[EMB_DIM : 2 * EMB_DIM].T,
        W1[2 * EMB_DIM :].T,
        b1.reshape(1, HIDDEN),
        W2.T,
        b2.reshape(1, HIDDEN),
        W3.T,
        b3.reshape(1, OUT_DIM),
    )
    return out_t.T
